# baseline (speedup 1.0000x reference)
import functools

import jax
import jax.numpy as jnp
from jax import lax
from jax.experimental import pallas as pl
from jax.experimental.pallas import tpu as pltpu
from jax.experimental.pallas import tpu_sc as plsc

ALPHA = 0.75
GAMMA = 2.0
NEG_RATIO = 3
NEG_PER_NEG_SLICE = 10

_BLOCK_ROWS = 8
_L = 16
_NW = 32
_HBITS = 19
_SBITS = 7


def _tc_body(logits_ref, labels_ref, score_ref, psum_ref, pcnt_ref, k_ref):
    l = logits_ref[...]
    y = labels_ref[...]
    valid = (y >= 0.0).astype(jnp.float32)
    t = jnp.clip(y, 0.0, None)
    bce = jnp.maximum(l, 0.0) - l * t + jnp.log1p(jnp.exp(-jnp.abs(l)))
    p = jax.nn.sigmoid(l)
    pt = t * p + (1.0 - t) * (1.0 - p)
    one_m_pt = 1.0 - pt
    alpha_w = t * ALPHA + (1.0 - t) * (1.0 - ALPHA)
    pel = alpha_w * one_m_pt * one_m_pt * bce * valid

    pos = y == 1.0
    neg = y == 0.0
    psum_ref[...] = jnp.sum(jnp.where(pos, pel, 0.0), axis=1, keepdims=True)
    pcnt_ref[...] = jnp.sum(jnp.where(pos & (pel > 0.0), 1.0, 0.0), axis=1,
                            keepdims=True)
    n_pos = jnp.sum(jnp.where(pos, 1, 0).astype(jnp.int32), axis=1, keepdims=True)
    n_keep = jnp.where(
        n_pos > 0,
        jnp.maximum(1, n_pos * jnp.int32(NEG_RATIO)),
        jnp.int32(NEG_PER_NEG_SLICE),
    )
    score = jnp.where(neg & (pel > 0.0), pel, -1.0)
    score_ref[...] = score
    cnt_posneg = jnp.sum(jnp.where(score > 0.0, 1, 0).astype(jnp.int32), axis=1,
                         keepdims=True)
    k_ref[...] = jnp.minimum(n_keep, cnt_posneg)


def _sc_body(score_hbm, k_hbm, out_hbm, data_v, h1_v, h2c_v, h2s_v, kbuf_v,
             out_v, sem):
    cid = lax.axis_index("c")
    sid = lax.axis_index("s")
    wid = sid * 2 + cid
    lane = lax.iota(jnp.int32, 16)
    flane = lane.astype(jnp.float32)

    kbase = (wid // 4) * 16
    pltpu.sync_copy(k_hbm.at[pl.ds(kbase, 16)], kbuf_v)
    kvec = kbuf_v[...].astype(jnp.float32)

    res = jnp.zeros((16,), jnp.float32)
    for r in range(4):
        row = wid * 4 + r
        j_in_block = (wid % 4) * 4 + r
        kf = jnp.sum(jnp.where(lane == j_in_block, kvec, 0.0))

        pltpu.async_copy(score_hbm.at[row], data_v, sem).wait()

        def _zero(i, _):
            z = jnp.zeros((16,), jnp.float32)
            h1_v[pl.ds(i * 16, 16)] = z
            h2c_v[pl.ds(i * 16, 16)] = z
            h2s_v[pl.ds(i * 16, 16)] = z
            return 0

        lax.fori_loop(0, 256, _zero, 0)

        def _p1(i, _):
            v = data_v[pl.ds(i * 16, 16)]
            bits = plsc.bitcast(v, jnp.int32)
            msk = bits > 0
            idx = jnp.where(msk, lax.shift_right_arithmetic(bits, _HBITS), 0)
            plsc.addupdate_scatter(h1_v, [idx],
                                   jnp.where(msk, 1.0, 0.0))
            return 0

        lax.fori_loop(0, 2048, _p1, 0)

        def _s1(tt, carry):
            found, b_star, cnt_above, cum = carry
            j = 255 - tt
            c = lax.rev(h1_v[pl.ds(j * 16, 16)], (0,))
            csum = plsc.cumsum(c) + cum
            f = jnp.sum(jnp.where(csum < kf, 1.0, 0.0))
            fi = f.astype(jnp.int32)
            hit = jnp.logical_and(found == 0, fi < 16)
            add_above = jnp.sum(jnp.where(lane < fi, c, 0.0))
            found = jnp.where(hit, 1, found)
            b_star = jnp.where(hit, j * 16 + (15 - fi), b_star)
            cnt_above = jnp.where(hit, cum + add_above, cnt_above)
            cum = cum + jnp.sum(c)
            return found, b_star, cnt_above, cum

        _, b_star, cnt_above, _ = lax.fori_loop(
            0, 256, _s1, (jnp.int32(0), jnp.int32(0), 0.0, 0.0))

        def _p2(i, acc):
            v = data_v[pl.ds(i * 16, 16)]
            bits = plsc.bitcast(v, jnp.int32)
            msk = bits > 0
            b = jnp.where(msk, lax.shift_right_arithmetic(bits, _HBITS), -1)
            acc = acc + jnp.where(b > b_star, v, 0.0)
            inb = b == b_star
            sub = jnp.where(
                inb,
                jnp.bitwise_and(lax.shift_right_arithmetic(bits, _SBITS), 0xFFF),
                0)
            w = jnp.where(inb, 1.0, 0.0)
            plsc.addupdate_scatter(h2c_v, [sub], w)
            plsc.addupdate_scatter(h2s_v, [sub], jnp.where(inb, v, 0.0))
            return acc

        acc = lax.fori_loop(0, 2048, _p2, jnp.zeros((16,), jnp.float32))
        sum_above = jnp.sum(acc)
        k2 = kf - cnt_above

        def _s2b(tt, carry):
            found, cnt_ab2, sum_ab2, bc, bs, cum, cums = carry
            j = 255 - tt
            c = lax.rev(h2c_v[pl.ds(j * 16, 16)], (0,))
            s = lax.rev(h2s_v[pl.ds(j * 16, 16)], (0,))
            csum = plsc.cumsum(c) + cum
            f = jnp.sum(jnp.where(csum < k2, 1.0, 0.0))
            fi = f.astype(jnp.int32)
            hit = jnp.logical_and(found == 0, fi < 16)
            add_above = jnp.sum(jnp.where(lane < fi, c, 0.0))
            add_sabove = jnp.sum(jnp.where(lane < fi, s, 0.0))
            bc_lane = jnp.sum(jnp.where(lane == fi, c, 0.0))
            bs_lane = jnp.sum(jnp.where(lane == fi, s, 0.0))
            return (jnp.where(hit, 1, found),
                    jnp.where(hit, cum + add_above, cnt_ab2),
                    jnp.where(hit, cums + add_sabove, sum_ab2),
                    jnp.where(hit, bc_lane, bc),
                    jnp.where(hit, bs_lane, bs),
                    cum + jnp.sum(c),
                    cums + jnp.sum(s))

        _, cnt_ab2, sum_ab2, bc, bs, _, _ = lax.fori_loop(
            0, 256, _s2b, (jnp.int32(0), 0.0, 0.0, 0.0, 0.0, 0.0, 0.0))

        k3 = k2 - cnt_ab2
        mean_vec = (jnp.full((16,), bs, jnp.float32)
                    / jnp.maximum(jnp.full((16,), bc, jnp.float32), 1.0))
        kept_vec = (sum_above + sum_ab2) + k3 * mean_vec
        have = kf > 0.0
        res = jnp.where(jnp.logical_and(lane == r, have), kept_vec, res)
        res = jnp.where(jnp.logical_and(lane == 4 + r, have),
                        jnp.full((16,), kf, jnp.float32), res)

    out_v[...] = res
    pltpu.sync_copy(out_v, out_hbm.at[wid])


def _sc_select(score, kvec):
    mesh = plsc.VectorSubcoreMesh(core_axis_name="c", subcore_axis_name="s",
                                  num_cores=2, num_subcores=16)
    f = functools.partial(
        pl.kernel,
        out_type=jax.ShapeDtypeStruct((_NW, 16), jnp.float32),
        mesh=mesh,
        compiler_params=pltpu.CompilerParams(needs_layout_passes=False),
        scratch_types=[
            pltpu.VMEM((32768,), jnp.float32),
            pltpu.VMEM((4096,), jnp.float32),
            pltpu.VMEM((4096,), jnp.float32),
            pltpu.VMEM((4096,), jnp.float32),
            pltpu.VMEM((16,), jnp.int32),
            pltpu.VMEM((16,), jnp.float32),
            pltpu.SemaphoreType.DMA,
        ],
    )(_sc_body)
    return f(score, kvec)


def kernel(logits, labels):
    B, N = logits.shape
    logits = logits.astype(jnp.float32)
    labels = labels.astype(jnp.float32)
    grid = B // _BLOCK_ROWS
    score, psum, pcnt, kv = pl.pallas_call(
        _tc_body,
        grid=(grid,),
        in_specs=[
            pl.BlockSpec((_BLOCK_ROWS, N), lambda i: (i, 0)),
            pl.BlockSpec((_BLOCK_ROWS, N), lambda i: (i, 0)),
        ],
        out_specs=[
            pl.BlockSpec((_BLOCK_ROWS, N), lambda i: (i, 0)),
            pl.BlockSpec((_BLOCK_ROWS, 1), lambda i: (i, 0)),
            pl.BlockSpec((_BLOCK_ROWS, 1), lambda i: (i, 0)),
            pl.BlockSpec((_BLOCK_ROWS, 1), lambda i: (i, 0)),
        ],
        out_shape=[
            jax.ShapeDtypeStruct((B, N), jnp.float32),
            jax.ShapeDtypeStruct((B, 1), jnp.float32),
            jax.ShapeDtypeStruct((B, 1), jnp.float32),
            jax.ShapeDtypeStruct((B, 1), jnp.int32),
        ],
    )(logits, labels)

    sc_out = _sc_select(score, kv.reshape(B))
    total = jnp.sum(psum) + jnp.sum(sc_out[:, 0:4])
    n_valid = jnp.maximum(jnp.sum(pcnt) + jnp.sum(sc_out[:, 4:8]), 1.0)
    return total / n_valid
